# baseline (speedup 1.0000x reference)
import jax
import jax.numpy as jnp
from jax.experimental import pallas as pl
from jax.experimental.pallas import tpu as pltpu

K = 2
TAU = 0.01
MAX_ITER = 100
TOL = 1e-3

N_ROWS = 16384
SUB = 8
LANE = N_ROWS // SUB


def _sigmoid(x):
    return 0.5 * jnp.tanh(0.5 * x) + 0.5


def _select_third_largest(c):
    v0 = jnp.minimum(c[0], c[1]); v1 = jnp.maximum(c[0], c[1])
    v2 = jnp.minimum(c[2], c[3]); v3 = jnp.maximum(c[2], c[3])
    v4 = jnp.minimum(c[4], c[5]); v5 = jnp.maximum(c[4], c[5])
    v6 = jnp.minimum(c[6], c[7]); v7 = jnp.maximum(c[6], c[7])
    w2 = jnp.maximum(v0, v2)
    w1 = jnp.minimum(v1, v3); w3 = jnp.maximum(v1, v3)
    w6 = jnp.maximum(v4, v6)
    w5 = jnp.minimum(v5, v7); w7 = jnp.maximum(v5, v7)
    x1 = jnp.minimum(w1, w2); x2 = jnp.maximum(w1, w2)
    x5 = jnp.minimum(w5, w6); x6 = jnp.maximum(w5, w6)
    y5 = jnp.maximum(x1, x5)
    y6 = jnp.maximum(x2, x6)
    y3 = jnp.minimum(w3, w7)
    z5 = jnp.maximum(y3, y5)
    return jnp.minimum(z5, y6)


def _gate_kernel(vt_ref, g_ref, st_s):
    st_s[...] = (vt_ref[...].reshape(16, 8, LANE // 2)
                 .transpose(1, 0, 2)
                 .reshape(8, SUB, LANE)) / TAU
    st = st_s[...]

    theta0 = _select_third_largest([st[j] for j in range(8)])


    sts = st
    t0 = theta0

    def body(carry):
        theta, i, done = carry
        sig = _sigmoid(sts - theta[None])
        f = jnp.sum(sig, axis=0) - K
        new_done = (jnp.sum(f) / N_ROWS) < TOL
        w = jnp.sum(sig * (1.0 - sig), axis=0)
        theta_new = theta + f / w
        theta_out = jnp.where(new_done, theta, theta_new)
        return (theta_out, i + 1, new_done)

    def cond(carry):
        _, i, done = carry
        return jnp.logical_and(i < MAX_ITER, jnp.logical_not(done))

    theta, _, _ = jax.lax.while_loop(
        cond, body, (t0, jnp.int32(0), jnp.bool_(False))
    )

    g = _sigmoid(sts - theta[None])
    g_ref[...] = (g.reshape(8, 16, LANE // 2)
                  .transpose(1, 0, 2)
                  .reshape(128, LANE // 2))


@jax.jit
def kernel(s):
    vt = s.reshape(1024, 128).T
    g_t = pl.pallas_call(
        _gate_kernel,
        out_shape=jax.ShapeDtypeStruct((128, 1024), s.dtype),
        in_specs=[pl.BlockSpec(memory_space=pltpu.VMEM)],
        out_specs=pl.BlockSpec(memory_space=pltpu.VMEM),
        scratch_shapes=[pltpu.VMEM((8, SUB, LANE), jnp.float32)],
    )(vt)
    return g_t.T.reshape(N_ROWS, 8)


# speedup vs baseline: 7.3186x; 7.3186x over previous
import jax
import jax.numpy as jnp
from jax.experimental import pallas as pl
from jax.experimental.pallas import tpu as pltpu

K = 2
TAU = 0.01
MAX_ITER = 100
TOL = 1e-3

N_ROWS = 16384
SUB = 8
LANE = N_ROWS // SUB


def _sigmoid(x):
    return 0.5 * jnp.tanh(0.5 * x) + 0.5


def _select_third_largest(c):
    v0 = jnp.minimum(c[0], c[1]); v1 = jnp.maximum(c[0], c[1])
    v2 = jnp.minimum(c[2], c[3]); v3 = jnp.maximum(c[2], c[3])
    v4 = jnp.minimum(c[4], c[5]); v5 = jnp.maximum(c[4], c[5])
    v6 = jnp.minimum(c[6], c[7]); v7 = jnp.maximum(c[6], c[7])
    w2 = jnp.maximum(v0, v2)
    w1 = jnp.minimum(v1, v3); w3 = jnp.maximum(v1, v3)
    w6 = jnp.maximum(v4, v6)
    w5 = jnp.minimum(v5, v7); w7 = jnp.maximum(v5, v7)
    x1 = jnp.minimum(w1, w2); x2 = jnp.maximum(w1, w2)
    x5 = jnp.minimum(w5, w6); x6 = jnp.maximum(w5, w6)
    y5 = jnp.maximum(x1, x5)
    y6 = jnp.maximum(x2, x6)
    y3 = jnp.minimum(w3, w7)
    z5 = jnp.maximum(y3, y5)
    return jnp.minimum(z5, y6)


def _gate_kernel(st_ref, g_ref):
    st = st_ref[...].reshape(8, SUB, LANE)

    theta0 = _select_third_largest([st[j] for j in range(8)])

    sts = st / TAU
    t0 = theta0 / TAU

    def body(carry):
        theta, i, done = carry
        sig = _sigmoid(sts - theta[None])
        f = jnp.sum(sig, axis=0) - K
        new_done = (jnp.sum(f) / N_ROWS) < TOL
        w = jnp.sum(sig * (1.0 - sig), axis=0)
        theta_new = theta + f / w
        theta_out = jnp.where(new_done, theta, theta_new)
        return (theta_out, i + 1, new_done)

    def cond(carry):
        _, i, done = carry
        return jnp.logical_and(i < MAX_ITER, jnp.logical_not(done))

    theta, _, _ = jax.lax.while_loop(
        cond, body, (t0, jnp.int32(0), jnp.bool_(False))
    )

    g = _sigmoid(sts - theta[None])
    g_ref[...] = g.reshape(8, N_ROWS)


@jax.jit
def kernel(s):
    st = s.T
    g_t = pl.pallas_call(
        _gate_kernel,
        out_shape=jax.ShapeDtypeStruct(st.shape, st.dtype),
        in_specs=[pl.BlockSpec(memory_space=pltpu.VMEM)],
        out_specs=pl.BlockSpec(memory_space=pltpu.VMEM),
    )(st)
    return g_t.T
